# baseline (device time: 148341 ns/iter reference)
import jax
import jax.numpy as jnp
from jax import lax
from jax.experimental import pallas as pl
from jax.experimental.pallas import tpu as pltpu

N_DEV = 32
HR = N_DEV // 2
HL = N_DEV // 2 - 1
SUB = 8


def _build_ring_tables():
    ident = list(range(N_DEV))
    try:
        coords = sorted(
            tuple(d.coords) for d in jax.devices()
            if getattr(d, "core_on_chip", 1) == 1
        )
    except Exception:
        return ident, ident
    if len(coords) != N_DEV or len(coords[0]) != 3:
        return ident, ident
    xs = sorted({c[0] for c in coords})
    ys = sorted({c[1] for c in coords})
    zs = sorted({c[2] for c in coords})
    full = {(x, y, z) for x in xs for y in ys for z in zs}
    if len(xs) != 2 or len(zs) % 2 or full != set(coords):
        return ident, ident

    ring = []
    for z in zs:
        plane = sorted(c for c in coords if c[2] == z)
        for yi, y in enumerate(ys):
            ring.extend(sorted((c for c in plane if c[1] == y),
                               reverse=bool(yi % 2)))
    logical = {c: i for i, c in enumerate(ring)}

    path_yz = []
    for zi, z in enumerate(zs):
        path_yz.extend((y, z) for y in (ys if zi % 2 == 0 else ys[::-1]))
    ham_coords = [(xs[1], y, z) for (y, z) in path_yz]
    ham_coords += [(xs[0], y, z) for (y, z) in path_yz[::-1]]

    for a, b in zip(ham_coords, ham_coords[1:] + ham_coords[:1]):
        if sum(abs(i - j) for i, j in zip(a, b)) != 1:
            return ident, ident
    ham = [logical[c] for c in ham_coords]
    pos = [0] * N_DEV
    for p, d in enumerate(ham):
        pos[d] = p
    return ham, pos


_HAM, _POS = _build_ring_tables()


def _lookup(table, idx):
    acc = jnp.int32(table[0])
    for j in range(1, N_DEV):
        acc = jnp.where(idx == j, jnp.int32(table[j]), acc)
    return acc


def kernel(x, w_mat, scale_x, scale_w):
    m_per, k = x.shape
    _, n_per = w_mat.shape
    s = (scale_x * scale_w).reshape(1, 1).astype(jnp.float32)

    def body(s_ref, x_ref, w_ref, out_ref,
             xs_ref, cr_ref, cl_ref, ssR, srR, ssL, srL):
        me = lax.axis_index("i")
        r = _lookup(_POS, me)
        nbrR = _lookup(_HAM, (r + 1) % N_DEV)
        nbrL = _lookup(_HAM, (r + N_DEV - 1) % N_DEV)

        xs_ref[...] = x_ref[...].astype(jnp.float8_e5m2)

        barrier = pltpu.get_barrier_semaphore()
        for nbr in (nbrL, nbrR):
            pl.semaphore_signal(
                barrier, inc=1,
                device_id=(nbr,), device_id_type=pl.DeviceIdType.MESH,
            )
        pl.semaphore_wait(barrier, 2)

        w_bf = w_ref[...].astype(jnp.bfloat16)
        s_val = s_ref[0, 0]

        def store_chunk(origin, chunk_e5m2):
            acc = jnp.dot(chunk_e5m2.astype(jnp.bfloat16), w_bf,
                          preferred_element_type=jnp.float32)
            out_ref[pl.ds(origin * m_per, m_per), :] = (
                jnp.maximum(acc * s_val, 0.0))

        msub = m_per // SUB

        def mk(h, si, comm_ref, ss, sr, nbr):
            rows = pl.ds(si * msub, msub)
            return pltpu.make_async_remote_copy(
                src_ref=(xs_ref.at[rows]
                         if h == 0 else comm_ref.at[h - 1, rows]),
                dst_ref=comm_ref.at[h, rows],
                send_sem=ss.at[h, si], recv_sem=sr.at[h, si],
                device_id=(nbr,), device_id_type=pl.DeviceIdType.MESH,
            )

        def mkR(h, si):
            return mk(h, si, cr_ref, ssR, srR, nbrR)

        def mkL(h, si):
            return mk(h, si, cl_ref, ssL, srL, nbrL)

        descR = [[mkR(0, si) for si in range(SUB)]]
        descL = [[mkL(0, si) for si in range(SUB)]]
        for si in range(SUB):
            descR[0][si].start()
            descL[0][si].start()

        store_chunk(me, xs_ref[...])

        for h in range(HR):
            if h + 1 < HR:
                descR.append([mkR(h + 1, si) for si in range(SUB)])
            if h + 1 < HL:
                descL.append([mkL(h + 1, si) for si in range(SUB)])
            for si in range(SUB):
                descR[h][si].wait_recv()
                if h + 1 < HR:
                    descR[h + 1][si].start()
                if h < HL:
                    descL[h][si].wait_recv()
                    if h + 1 < HL:
                        descL[h + 1][si].start()
            store_chunk(_lookup(_HAM, (r + N_DEV - h - 1) % N_DEV), cr_ref[h])
            if h < HL:
                store_chunk(_lookup(_HAM, (r + h + 1) % N_DEV), cl_ref[h])

        for hop in descR + descL:
            for d in hop:
                d.wait_send()

    return pl.pallas_call(
        body,
        out_shape=jax.ShapeDtypeStruct((N_DEV * m_per, n_per), jnp.float32),
        in_specs=[
            pl.BlockSpec(memory_space=pltpu.SMEM),
            pl.BlockSpec(memory_space=pltpu.VMEM),
            pl.BlockSpec(memory_space=pltpu.VMEM),
        ],
        out_specs=pl.BlockSpec(memory_space=pltpu.VMEM),
        scratch_shapes=[
            pltpu.VMEM((m_per, k), jnp.float8_e5m2),
            pltpu.VMEM((HR, m_per, k), jnp.float8_e5m2),
            pltpu.VMEM((HL, m_per, k), jnp.float8_e5m2),
            pltpu.SemaphoreType.DMA((HR, SUB)),
            pltpu.SemaphoreType.DMA((HR, SUB)),
            pltpu.SemaphoreType.DMA((HL, SUB)),
            pltpu.SemaphoreType.DMA((HL, SUB)),
        ],
        compiler_params=pltpu.CompilerParams(collective_id=0),
    )(s, x, w_mat)


# device time: 100781 ns/iter; 1.4719x vs baseline; 1.4719x over previous
import jax
import jax.numpy as jnp
from jax import lax
from jax.experimental import pallas as pl
from jax.experimental.pallas import tpu as pltpu

N_DEV = 32
HR = N_DEV // 2
HL = N_DEV // 2 - 1
SUB = 4


def _build_ring_tables():
    ident = list(range(N_DEV))
    try:
        coords = sorted(
            tuple(d.coords) for d in jax.devices()
            if getattr(d, "core_on_chip", 1) == 1
        )
    except Exception:
        return ident, ident
    if len(coords) != N_DEV or len(coords[0]) != 3:
        return ident, ident
    xs = sorted({c[0] for c in coords})
    ys = sorted({c[1] for c in coords})
    zs = sorted({c[2] for c in coords})
    full = {(x, y, z) for x in xs for y in ys for z in zs}
    if len(xs) != 2 or len(zs) % 2 or full != set(coords):
        return ident, ident

    ring = []
    for z in zs:
        plane = sorted(c for c in coords if c[2] == z)
        for yi, y in enumerate(ys):
            ring.extend(sorted((c for c in plane if c[1] == y),
                               reverse=bool(yi % 2)))
    logical = {c: i for i, c in enumerate(ring)}

    path_yz = []
    for zi, z in enumerate(zs):
        path_yz.extend((y, z) for y in (ys if zi % 2 == 0 else ys[::-1]))
    ham_coords = [(xs[1], y, z) for (y, z) in path_yz]
    ham_coords += [(xs[0], y, z) for (y, z) in path_yz[::-1]]

    for a, b in zip(ham_coords, ham_coords[1:] + ham_coords[:1]):
        if sum(abs(i - j) for i, j in zip(a, b)) != 1:
            return ident, ident
    ham = [logical[c] for c in ham_coords]
    pos = [0] * N_DEV
    for p, d in enumerate(ham):
        pos[d] = p
    return ham, pos


_HAM, _POS = _build_ring_tables()


def _lookup(table, idx):
    acc = jnp.int32(table[0])
    for j in range(1, N_DEV):
        acc = jnp.where(idx == j, jnp.int32(table[j]), acc)
    return acc


def kernel(x, w_mat, scale_x, scale_w):
    m_per, k = x.shape
    _, n_per = w_mat.shape
    s = (scale_x * scale_w).reshape(1, 1).astype(jnp.float32)

    def body(s_ref, x_ref, w_ref, out_ref,
             xs_ref, cr_ref, cl_ref, ssR, srR, ssL, srL):
        me = lax.axis_index("i")
        r = _lookup(_POS, me)
        nbrR = _lookup(_HAM, (r + 1) % N_DEV)
        nbrL = _lookup(_HAM, (r + N_DEV - 1) % N_DEV)

        barrier = pltpu.get_barrier_semaphore()
        for nbr in (nbrL, nbrR):
            pl.semaphore_signal(
                barrier, inc=1,
                device_id=(nbr,), device_id_type=pl.DeviceIdType.MESH,
            )
        xs_ref[...] = x_ref[...].astype(jnp.float8_e5m2)
        pl.semaphore_wait(barrier, 2)

        w_q = w_ref[...].astype(jnp.float8_e5m2)
        s_val = s_ref[0, 0]

        def store_chunk(origin, chunk_e5m2):
            acc = jnp.dot(chunk_e5m2, w_q,
                          preferred_element_type=jnp.float32)
            out_ref[pl.ds(origin * m_per, m_per), :] = (
                jnp.maximum(acc * s_val, 0.0))

        msub = m_per // SUB

        def mk(h, si, comm_ref, ss, sr, nbr):
            rows = pl.ds(si * msub, msub)
            return pltpu.make_async_remote_copy(
                src_ref=(xs_ref.at[rows]
                         if h == 0 else comm_ref.at[h - 1, rows]),
                dst_ref=comm_ref.at[h, rows],
                send_sem=ss.at[h, si], recv_sem=sr.at[h, si],
                device_id=(nbr,), device_id_type=pl.DeviceIdType.MESH,
            )

        def mkR(h, si):
            return mk(h, si, cr_ref, ssR, srR, nbrR)

        def mkL(h, si):
            return mk(h, si, cl_ref, ssL, srL, nbrL)

        descR = [[mkR(0, si) for si in range(SUB)]]
        descL = [[mkL(0, si) for si in range(SUB)]]
        for si in range(SUB):
            descR[0][si].start()
            descL[0][si].start()

        store_chunk(me, xs_ref[...])

        for h in range(HR):
            if h + 1 < HR:
                descR.append([mkR(h + 1, si) for si in range(SUB)])
            if h + 1 < HL:
                descL.append([mkL(h + 1, si) for si in range(SUB)])
            for si in range(SUB):
                descR[h][si].wait_recv()
                if h + 1 < HR:
                    descR[h + 1][si].start()
                if h < HL:
                    descL[h][si].wait_recv()
                    if h + 1 < HL:
                        descL[h + 1][si].start()
            store_chunk(_lookup(_HAM, (r + N_DEV - h - 1) % N_DEV), cr_ref[h])
            if h < HL:
                store_chunk(_lookup(_HAM, (r + h + 1) % N_DEV), cl_ref[h])

        for hop in descR + descL:
            for d in hop:
                d.wait_send()

    return pl.pallas_call(
        body,
        out_shape=jax.ShapeDtypeStruct((N_DEV * m_per, n_per), jnp.float32),
        in_specs=[
            pl.BlockSpec(memory_space=pltpu.SMEM),
            pl.BlockSpec(memory_space=pltpu.VMEM),
            pl.BlockSpec(memory_space=pltpu.VMEM),
        ],
        out_specs=pl.BlockSpec(memory_space=pltpu.VMEM),
        scratch_shapes=[
            pltpu.VMEM((m_per, k), jnp.float8_e5m2),
            pltpu.VMEM((HR, m_per, k), jnp.float8_e5m2),
            pltpu.VMEM((HL, m_per, k), jnp.float8_e5m2),
            pltpu.SemaphoreType.DMA((HR, SUB)),
            pltpu.SemaphoreType.DMA((HR, SUB)),
            pltpu.SemaphoreType.DMA((HL, SUB)),
            pltpu.SemaphoreType.DMA((HL, SUB)),
        ],
        compiler_params=pltpu.CompilerParams(collective_id=0),
    )(s, x, w_mat)
